# baseline (speedup 1.0000x reference)
import jax
import jax.numpy as jnp
from jax import lax
from jax.experimental import pallas as pl
from jax.experimental.pallas import tpu as pltpu
from jax.experimental.pallas import tpu_sc as plsc

_SEQ = 8192
_NC = 2
_NS = 16
_L = 16
_NW = _NC * _NS
_OUT = _SEQ // _NW


def _lane_reduce(v, buf, op):
    for sh in (8, 4, 2, 1):
        buf[pl.ds(0, _L)] = v
        buf[pl.ds(_L, _L)] = v
        v = op(v, buf[pl.ds(sh, _L)])
    return v


def _gate_body(col0_hbm, col1_hbm, pos_hbm, out0_hbm, out1_hbm,
               a_v, b_v, pos_v, o0_v, o1_v, buf_v):
    c = lax.axis_index("c")
    s = lax.axis_index("s")
    wid = s * _NC + c

    pltpu.sync_copy(col0_hbm, a_v)
    pltpu.sync_copy(col1_hbm, b_v)
    pltpu.sync_copy(pos_hbm, pos_v)

    big = jnp.full((_L,), 3.4e38, jnp.float32)

    def red_step(i, mm):
        mn, mx = mm
        d = a_v[pl.ds(i * _L, _L)] - b_v[pl.ds(i * _L, _L)]
        return jnp.minimum(mn, d), jnp.maximum(mx, d)

    mn, mx = lax.fori_loop(0, _SEQ // _L, red_step, (big, -big))
    mn = _lane_reduce(mn, buf_v, jnp.minimum)
    mx = _lane_reduce(mx, buf_v, jnp.maximum)
    one = jnp.full((_L,), 1.0, jnp.float32)
    zero = jnp.full((_L,), 0.0, jnp.float32)
    cond_f = jnp.where(mn >= zero, one, zero) + jnp.where(mx < zero, one, zero)

    pv = pos_v[...]
    out_base = wid * _OUT
    lane = lax.iota(jnp.int32, _L)
    for j in range(_OUT // _L):
        a = a_v[pl.ds(out_base + j * _L, _L)]
        b = b_v[pl.ds(out_base + j * _L, _L)]
        h = jnp.where(a >= b, one, zero)
        rows = out_base + j * _L + lane
        flip = cond_f * jnp.where(rows == pv, one, zero)
        sel = h + flip - 2.0 * h * flip
        o0_v[pl.ds(j * _L, _L)] = sel
        o1_v[pl.ds(j * _L, _L)] = one - sel
    pltpu.sync_copy(o0_v, out0_hbm.at[pl.ds(out_base, _OUT)])
    pltpu.sync_copy(o1_v, out1_hbm.at[pl.ds(out_base, _OUT)])


@jax.jit
def _gate_sc(col0, col1, pos_arr):
    mesh = plsc.VectorSubcoreMesh(core_axis_name="c", subcore_axis_name="s",
                                  num_cores=_NC, num_subcores=_NS)
    f32 = jnp.float32
    run = pl.kernel(
        _gate_body,
        out_type=(jax.ShapeDtypeStruct((_SEQ,), f32),
                  jax.ShapeDtypeStruct((_SEQ,), f32)),
        mesh=mesh,
        scratch_types=[
            pltpu.VMEM((_SEQ,), f32),
            pltpu.VMEM((_SEQ,), f32),
            pltpu.VMEM((_L,), jnp.int32),
            pltpu.VMEM((_OUT,), f32),
            pltpu.VMEM((_OUT,), f32),
            pltpu.VMEM((2 * _L,), f32),
        ],
        name="me_gated_balance_mask",
    )
    return run(col0, col1, pos_arr)


def kernel(x, W, global_gate_score):
    del x, W
    ggs = global_gate_score
    col0 = ggs[:, 0]
    col1 = ggs[:, 1]
    pos = jax.random.randint(jax.random.key(42), (), 0, ggs.shape[0])
    pos_arr = jnp.broadcast_to(pos.astype(jnp.int32), (_L,))
    out0, out1 = _gate_sc(col0, col1, pos_arr)
    return (out0, out1)


# speedup vs baseline: 1.5473x; 1.5473x over previous
import jax
import jax.numpy as jnp
from jax import lax
from jax.experimental import pallas as pl
from jax.experimental.pallas import tpu as pltpu
from jax.experimental.pallas import tpu_sc as plsc

_SEQ = 8192
_NS = 16
_L = 16
_BLK = _SEQ // _NS
_UNROLL = 8

_POS = int(jax.random.randint(jax.random.key(42), (), 0, _SEQ))


def _lane_reduce(v, buf, op):
    for sh in (8, 4, 2, 1):
        buf[pl.ds(0, _L)] = v
        buf[pl.ds(_L, _L)] = v
        v = op(v, buf[pl.ds(sh, _L)])
    return v


def _make_body(pos):
    owner = pos // _BLK

    def _gate_body(col0_hbm, col1_hbm, out0_hbm, out1_hbm,
                   a_v, b_v, ga_v, gb_v, o0_v, o1_v, buf_v, cond_v):
        s = lax.axis_index("s")
        base = s * _BLK
        one = jnp.full((_L,), 1.0, jnp.float32)
        zero = jnp.full((_L,), 0.0, jnp.float32)

        pltpu.sync_copy(col0_hbm.at[pl.ds(base, _BLK)], a_v)
        pltpu.sync_copy(col1_hbm.at[pl.ds(base, _BLK)], b_v)

        cond_v[...] = zero

        @pl.when(s == owner)
        def _():
            pltpu.sync_copy(col0_hbm, ga_v)
            pltpu.sync_copy(col1_hbm, gb_v)
            big = jnp.full((_L,), 3.4e38, jnp.float32)

            def red_step(i, mm):
                mn, mx = mm
                for u in range(_UNROLL):
                    off = (i * _UNROLL + u) * _L
                    d = ga_v[pl.ds(off, _L)] - gb_v[pl.ds(off, _L)]
                    mn = jnp.minimum(mn, d)
                    mx = jnp.maximum(mx, d)
                return mn, mx

            mn, mx = lax.fori_loop(0, _SEQ // (_L * _UNROLL), red_step,
                                   (big, -big))
            mn = _lane_reduce(mn, buf_v, jnp.minimum)
            mx = _lane_reduce(mx, buf_v, jnp.maximum)
            cond_v[...] = (jnp.where(mn >= zero, one, zero)
                           + jnp.where(mx < zero, one, zero))

        cond_f = cond_v[...]

        lane = lax.iota(jnp.int32, _L)
        for j in range(_BLK // _L):
            a = a_v[pl.ds(j * _L, _L)]
            b = b_v[pl.ds(j * _L, _L)]
            h = jnp.where(a >= b, one, zero)
            rows = base + j * _L + lane
            flip = cond_f * jnp.where(rows == pos, one, zero)
            sel = h + flip - 2.0 * h * flip
            o0_v[pl.ds(j * _L, _L)] = sel
            o1_v[pl.ds(j * _L, _L)] = one - sel
        pltpu.sync_copy(o0_v, out0_hbm.at[pl.ds(base, _BLK)])
        pltpu.sync_copy(o1_v, out1_hbm.at[pl.ds(base, _BLK)])

    return _gate_body


@jax.jit
def _gate_sc(col0, col1):
    pos = _POS
    mesh = plsc.VectorSubcoreMesh(core_axis_name="c", subcore_axis_name="s",
                                  num_cores=1, num_subcores=_NS)
    f32 = jnp.float32
    run = pl.kernel(
        _make_body(pos),
        out_type=(jax.ShapeDtypeStruct((_SEQ,), f32),
                  jax.ShapeDtypeStruct((_SEQ,), f32)),
        mesh=mesh,
        scratch_types=[
            pltpu.VMEM((_BLK,), f32),
            pltpu.VMEM((_BLK,), f32),
            pltpu.VMEM((_SEQ,), f32),
            pltpu.VMEM((_SEQ,), f32),
            pltpu.VMEM((_BLK,), f32),
            pltpu.VMEM((_BLK,), f32),
            pltpu.VMEM((2 * _L,), f32),
            pltpu.VMEM((_L,), f32),
        ],
        name="me_gated_balance_mask",
    )
    return run(col0, col1)


def kernel(x, W, global_gate_score):
    del x, W
    ggs = global_gate_score
    out0, out1 = _gate_sc(ggs[:, 0], ggs[:, 1])
    return (out0, out1)
